# baseline (speedup 1.0000x reference)
import functools

import jax
import jax.numpy as jnp
from jax import lax
from jax.experimental import pallas as pl
from jax.experimental.pallas import tpu as pltpu
from jax.experimental.pallas import tpu_sc as plsc

E = 64
EP = 128
K = 8
D = 4096
T = 8192

BT = 512
BTS = BT // 2
NC = 2
CT = T // NC
NW = 32
TPW = CT // NW
NG = TPW // 16
L = 16


def _gate_body(x_ref, w1_ref, w2_ref, out_ref):
    h = jnp.tanh(jnp.dot(x_ref[...], w1_ref[...]))
    out_ref[...] = jnp.dot(h, w2_ref[...])


def _gate_logits(x, W1p, W2p, c):
    off = c * (CT // BT)
    return pl.pallas_call(
        _gate_body,
        grid=(CT // BT,),
        in_specs=[
            pl.BlockSpec((BT, D), lambda i: (off + i, 0)),
            pl.BlockSpec((D, EP), lambda i: (0, 0)),
            pl.BlockSpec((EP, EP), lambda i: (0, 0)),
        ],
        out_specs=pl.BlockSpec((BT, EP), lambda i: (i, 0)),
        out_shape=jax.ShapeDtypeStruct((CT, EP), jnp.float32),
    )(x, W1p, W2p)


def _topk_body(lg_hbm, oi_hbm, os_hbm, lg_v, oi_v, os_v):
    wid = lax.axis_index("s") * 2 + lax.axis_index("c")
    pltpu.sync_copy(lg_hbm.at[pl.ds(wid * TPW, TPW)], lg_v)

    lane = lax.iota(jnp.int32, L)

    def group_body(g, _):
        rows_ab = [2 * g * L + lane, (2 * g + 1) * L + lane]

        neg_inf = jnp.full((L,), -jnp.inf, jnp.float32)
        zero_i = jnp.zeros((L,), jnp.int32)
        init = tuple(neg_inf for _ in range(K)) + tuple(zero_i for _ in range(K))

        def expert_body(e, carry):
            ca, cb = carry
            iv0 = jnp.full((L,), 0, jnp.int32) + e
            out = []
            for rows, c in ((rows_ab[0], ca), (rows_ab[1], cb)):
                ts, is_ = c[:K], c[K:]
                v = plsc.load_gather(lg_v, [rows, iv0])
                iv = iv0
                ins = jnp.zeros((L,), jnp.bool_)
                new_ts, new_is = [], []
                for r in range(K):
                    gt = v > ts[r]
                    cond = jnp.logical_or(ins, gt)
                    new_ts.append(jnp.maximum(v, ts[r]))
                    v = jnp.minimum(v, ts[r])
                    new_is.append(jnp.where(cond, iv, is_[r]))
                    iv = jnp.where(cond, is_[r], iv)
                    ins = cond
                out.append(tuple(new_ts) + tuple(new_is))
            return tuple(out)

        ca, cb = lax.fori_loop(0, E, expert_body, (init, init))

        for rows, c in ((rows_ab[0], ca), (rows_ab[1], cb)):
            ts, is_ = c[:K], c[K:]
            rows8 = rows * K
            exps = [jnp.exp(t - ts[0]) for t in ts]
            s = exps[0]
            for r in range(1, K):
                s = s + exps[r]
            inv = 1.0 / s
            for r in range(K):
                plsc.store_scatter(oi_v, [rows8 + r], is_[r])
                plsc.store_scatter(os_v, [rows8 + r], exps[r] * inv)
        return _

    lax.fori_loop(0, NG // 2, group_body, None)

    obase = wid * (TPW * K)
    pltpu.sync_copy(oi_v, oi_hbm.at[pl.ds(obase, TPW * K)])
    pltpu.sync_copy(os_v, os_hbm.at[pl.ds(obase, TPW * K)])


@functools.cache
def _topk_sc():
    return pl.kernel(
        _topk_body,
        out_type=(
            jax.ShapeDtypeStruct((CT * K,), jnp.int32),
            jax.ShapeDtypeStruct((CT * K,), jnp.float32),
        ),
        mesh=plsc.VectorSubcoreMesh(core_axis_name="c", subcore_axis_name="s"),
        compiler_params=pltpu.CompilerParams(needs_layout_passes=False),
        scratch_types=[
            pltpu.VMEM((TPW, EP), jnp.float32),
            pltpu.VMEM((TPW * K,), jnp.int32),
            pltpu.VMEM((TPW * K,), jnp.float32),
        ],
    )


def kernel(x, W1, W2):
    W1p = jnp.pad(W1, ((0, 0), (0, EP - E)))
    W2p = jnp.pad(W2, ((0, EP - E), (0, EP - E)))
    topk = _topk_sc()
    idx_parts, scr_parts = [], []
    for c in range(NC):
        logits = _gate_logits(x, W1p, W2p, c)
        idx_flat, scr_flat = topk(logits)
        idx_parts.append(idx_flat)
        scr_parts.append(scr_flat)
    idx = jnp.concatenate(idx_parts, 0).reshape(T, K)
    scr = jnp.concatenate(scr_parts, 0).reshape(T, K)
    return idx, scr


# speedup vs baseline: 1.0017x; 1.0017x over previous
import functools

import jax
import jax.numpy as jnp
from jax import lax
from jax.experimental import pallas as pl
from jax.experimental.pallas import tpu as pltpu
from jax.experimental.pallas import tpu_sc as plsc

E = 64
EP = 128
K = 8
D = 4096
T = 8192

BT = 512
BTS = BT // 2
NC = 2
CT = T // NC
NW = 32
TPW = CT // NW
NG = TPW // 16
L = 16


def _gate_body(x_ref, w1_ref, w2_ref, out_ref):
    h = jnp.tanh(jnp.dot(x_ref[...], w1_ref[...]))
    out_ref[...] = jnp.dot(h, w2_ref[...])


def _gate_logits(x, W1, W2p, c):
    off = c * (CT // BT)
    return pl.pallas_call(
        _gate_body,
        grid=(CT // BT,),
        in_specs=[
            pl.BlockSpec((BT, D), lambda i: (off + i, 0)),
            pl.BlockSpec((D, E), lambda i: (0, 0)),
            pl.BlockSpec((E, EP), lambda i: (0, 0)),
        ],
        out_specs=pl.BlockSpec((BT, EP), lambda i: (i, 0)),
        out_shape=jax.ShapeDtypeStruct((CT, EP), jnp.float32),
    )(x, W1, W2p)


def _topk_body(lg_hbm, oi_hbm, os_hbm, lg_v, oi_v, os_v):
    wid = lax.axis_index("s") * 2 + lax.axis_index("c")
    pltpu.sync_copy(lg_hbm.at[pl.ds(wid * TPW, TPW)], lg_v)

    lane = lax.iota(jnp.int32, L)

    def group_body(g, _):
        rows_ab = [2 * g * L + lane, (2 * g + 1) * L + lane]

        neg_inf = jnp.full((L,), -jnp.inf, jnp.float32)
        zero_i = jnp.zeros((L,), jnp.int32)
        init = tuple(neg_inf for _ in range(K)) + tuple(zero_i for _ in range(K))

        def expert_body(e, carry):
            ca, cb = carry
            iv0 = jnp.full((L,), 0, jnp.int32) + e
            out = []
            for rows, c in ((rows_ab[0], ca), (rows_ab[1], cb)):
                ts, is_ = c[:K], c[K:]
                v = plsc.load_gather(lg_v, [rows, iv0])
                iv = iv0
                ins = jnp.zeros((L,), jnp.bool_)
                new_ts, new_is = [], []
                for r in range(K):
                    gt = v > ts[r]
                    cond = jnp.logical_or(ins, gt)
                    new_ts.append(jnp.maximum(v, ts[r]))
                    v = jnp.minimum(v, ts[r])
                    new_is.append(jnp.where(cond, iv, is_[r]))
                    iv = jnp.where(cond, is_[r], iv)
                    ins = cond
                out.append(tuple(new_ts) + tuple(new_is))
            return tuple(out)

        ca, cb = lax.fori_loop(0, E, expert_body, (init, init))

        for rows, c in ((rows_ab[0], ca), (rows_ab[1], cb)):
            ts, is_ = c[:K], c[K:]
            rows8 = rows * K
            exps = [jnp.exp(t - ts[0]) for t in ts]
            s = exps[0]
            for r in range(1, K):
                s = s + exps[r]
            inv = 1.0 / s
            for r in range(K):
                plsc.store_scatter(oi_v, [rows8 + r], is_[r])
                plsc.store_scatter(os_v, [rows8 + r], exps[r] * inv)
        return _

    lax.fori_loop(0, NG // 2, group_body, None)

    obase = wid * (TPW * K)
    pltpu.sync_copy(oi_v, oi_hbm.at[pl.ds(obase, TPW * K)])
    pltpu.sync_copy(os_v, os_hbm.at[pl.ds(obase, TPW * K)])


@functools.cache
def _topk_sc():
    return pl.kernel(
        _topk_body,
        out_type=(
            jax.ShapeDtypeStruct((CT * K,), jnp.int32),
            jax.ShapeDtypeStruct((CT * K,), jnp.float32),
        ),
        mesh=plsc.VectorSubcoreMesh(core_axis_name="c", subcore_axis_name="s"),
        compiler_params=pltpu.CompilerParams(needs_layout_passes=False),
        scratch_types=[
            pltpu.VMEM((TPW, EP), jnp.float32),
            pltpu.VMEM((TPW * K,), jnp.int32),
            pltpu.VMEM((TPW * K,), jnp.float32),
        ],
    )


def kernel(x, W1, W2):
    W2p = jnp.pad(W2, ((0, 0), (0, EP - E)))
    topk = _topk_sc()
    idx_parts, scr_parts = [], []
    for c in range(NC):
        logits = _gate_logits(x, W1, W2p, c)
        idx_flat, scr_flat = topk(logits)
        idx_parts.append(idx_flat)
        scr_parts.append(scr_flat)
    idx = jnp.concatenate(idx_parts, 0).reshape(T, K)
    scr = jnp.concatenate(scr_parts, 0).reshape(T, K)
    return idx, scr


# speedup vs baseline: 1.0131x; 1.0114x over previous
import functools

import jax
import jax.numpy as jnp
from jax import lax
from jax.experimental import pallas as pl
from jax.experimental.pallas import tpu as pltpu
from jax.experimental.pallas import tpu_sc as plsc

E = 64
EP = 128
K = 8
D = 4096
T = 8192

BT = 512
BTS = BT // 2
NC = 2
CT = T // NC
NW = 32
TPW = CT // NW
NG = TPW // 16
L = 16


def _gate_body(x_ref, w1_ref, w2_ref, out_ref):
    h = jnp.tanh(jnp.dot(x_ref[...], w1_ref[...]))
    out_ref[...] = jnp.dot(h, w2_ref[...])


def _gate_topk_body(x_ref, w1_ref, w2_ref, oi_ref, os_ref):
    h = jnp.tanh(jnp.dot(x_ref[...], w1_ref[...]))
    lg = jnp.dot(h, w2_ref[...])
    cols = lax.broadcasted_iota(jnp.int32, (BT, E), 1)
    vals, idxs = [], []
    for r in range(K):
        m = jnp.max(lg, axis=1, keepdims=True)
        sel = jnp.min(jnp.where(lg == m, cols, E), axis=1, keepdims=True)
        vals.append(m)
        idxs.append(sel)
        lg = jnp.where(cols == sel, -jnp.inf, lg)
    exps = [jnp.exp(v - vals[0]) for v in vals]
    s = exps[0]
    for r in range(1, K):
        s = s + exps[r]
    inv = 1.0 / s
    oi_ref[...] = jnp.concatenate(idxs, axis=1)
    os_ref[...] = jnp.concatenate([e * inv for e in exps], axis=1)


def _gate_topk(x, W1, W2, c):
    off = c * (CT // BT)
    return pl.pallas_call(
        _gate_topk_body,
        grid=(CT // BT,),
        in_specs=[
            pl.BlockSpec((BT, D), lambda i: (off + i, 0)),
            pl.BlockSpec((D, E), lambda i: (0, 0)),
            pl.BlockSpec((E, E), lambda i: (0, 0)),
        ],
        out_specs=[
            pl.BlockSpec((BT, K), lambda i: (i, 0)),
            pl.BlockSpec((BT, K), lambda i: (i, 0)),
        ],
        out_shape=(
            jax.ShapeDtypeStruct((CT, K), jnp.int32),
            jax.ShapeDtypeStruct((CT, K), jnp.float32),
        ),
    )(x, W1, W2)


def _gate_logits(x, W1, W2p, c):
    off = c * (CT // BT)
    return pl.pallas_call(
        _gate_body,
        grid=(CT // BT,),
        in_specs=[
            pl.BlockSpec((BT, D), lambda i: (off + i, 0)),
            pl.BlockSpec((D, E), lambda i: (0, 0)),
            pl.BlockSpec((E, EP), lambda i: (0, 0)),
        ],
        out_specs=pl.BlockSpec((BT, EP), lambda i: (i, 0)),
        out_shape=jax.ShapeDtypeStruct((CT, EP), jnp.float32),
    )(x, W1, W2p)


def _topk_body(lg_hbm, oi_hbm, os_hbm, lg_v, oi_v, os_v):
    wid = lax.axis_index("s") * 2 + lax.axis_index("c")
    pltpu.sync_copy(lg_hbm.at[pl.ds(wid * TPW, TPW)], lg_v)

    lane = lax.iota(jnp.int32, L)

    def group_body(g, _):
        rows_ab = [2 * g * L + lane, (2 * g + 1) * L + lane]

        neg_inf = jnp.full((L,), -jnp.inf, jnp.float32)
        zero_i = jnp.zeros((L,), jnp.int32)
        init = tuple(neg_inf for _ in range(K)) + tuple(zero_i for _ in range(K))

        def expert_body(e, carry):
            ca, cb = carry
            iv0 = jnp.full((L,), 0, jnp.int32) + e
            out = []
            for rows, c in ((rows_ab[0], ca), (rows_ab[1], cb)):
                ts, is_ = c[:K], c[K:]
                v = plsc.load_gather(lg_v, [rows, iv0])
                iv = iv0
                ins = jnp.zeros((L,), jnp.bool_)
                new_ts, new_is = [], []
                for r in range(K):
                    gt = v > ts[r]
                    cond = jnp.logical_or(ins, gt)
                    new_ts.append(jnp.maximum(v, ts[r]))
                    v = jnp.minimum(v, ts[r])
                    new_is.append(jnp.where(cond, iv, is_[r]))
                    iv = jnp.where(cond, is_[r], iv)
                    ins = cond
                out.append(tuple(new_ts) + tuple(new_is))
            return tuple(out)

        ca, cb = lax.fori_loop(0, E, expert_body, (init, init))

        for rows, c in ((rows_ab[0], ca), (rows_ab[1], cb)):
            ts, is_ = c[:K], c[K:]
            rows8 = rows * K
            exps = [jnp.exp(t - ts[0]) for t in ts]
            s = exps[0]
            for r in range(1, K):
                s = s + exps[r]
            inv = 1.0 / s
            for r in range(K):
                plsc.store_scatter(oi_v, [rows8 + r], is_[r])
                plsc.store_scatter(os_v, [rows8 + r], exps[r] * inv)
        return _

    lax.fori_loop(0, NG // 2, group_body, None)

    obase = wid * (TPW * K)
    pltpu.sync_copy(oi_v, oi_hbm.at[pl.ds(obase, TPW * K)])
    pltpu.sync_copy(os_v, os_hbm.at[pl.ds(obase, TPW * K)])


@functools.cache
def _topk_sc():
    return pl.kernel(
        _topk_body,
        out_type=(
            jax.ShapeDtypeStruct((CT * K,), jnp.int32),
            jax.ShapeDtypeStruct((CT * K,), jnp.float32),
        ),
        mesh=plsc.VectorSubcoreMesh(core_axis_name="c", subcore_axis_name="s"),
        compiler_params=pltpu.CompilerParams(needs_layout_passes=False),
        scratch_types=[
            pltpu.VMEM((TPW, EP), jnp.float32),
            pltpu.VMEM((TPW * K,), jnp.int32),
            pltpu.VMEM((TPW * K,), jnp.float32),
        ],
    )


def kernel(x, W1, W2):
    W2p = jnp.pad(W2, ((0, 0), (0, EP - E)))
    logits0 = _gate_logits(x, W1, W2p, 0)
    idx0_flat, scr0_flat = _topk_sc()(logits0)
    idx1, scr1 = _gate_topk(x, W1, W2, 1)
    idx = jnp.concatenate([idx0_flat.reshape(CT, K), idx1], 0)
    scr = jnp.concatenate([scr0_flat.reshape(CT, K), scr1], 0)
    return idx, scr


# speedup vs baseline: 1.0297x; 1.0164x over previous
import functools

import jax
import jax.numpy as jnp
from jax import lax
from jax.experimental import pallas as pl
from jax.experimental.pallas import tpu as pltpu
from jax.experimental.pallas import tpu_sc as plsc

E = 64
EP = 128
K = 8
D = 4096
T = 8192

BT = 512
CT0 = 6144
CT1 = T - CT0
NW = 32
TPW = CT0 // NW
NG = TPW // 16
L = 16


def _gate_body(x_ref, w1_ref, w2_ref, out_ref):
    h = jnp.tanh(jnp.dot(x_ref[...], w1_ref[...]))
    out_ref[...] = jnp.dot(h, w2_ref[...])


def _gate_topk_body(x_ref, w1_ref, w2_ref, oi_ref, os_ref):
    h = jnp.tanh(jnp.dot(x_ref[...], w1_ref[...]))
    lg = jnp.dot(h, w2_ref[...])
    cols = lax.broadcasted_iota(jnp.int32, (BT, E), 1)
    vals, idxs = [], []
    for r in range(K):
        m = jnp.max(lg, axis=1, keepdims=True)
        sel = jnp.min(jnp.where(lg == m, cols, E), axis=1, keepdims=True)
        vals.append(m)
        idxs.append(sel)
        lg = jnp.where(cols == sel, -jnp.inf, lg)
    exps = [jnp.exp(v - vals[0]) for v in vals]
    s = exps[0]
    for r in range(1, K):
        s = s + exps[r]
    inv = 1.0 / s
    oi_ref[...] = jnp.concatenate(idxs, axis=1)
    os_ref[...] = jnp.concatenate([e * inv for e in exps], axis=1)


def _gate_topk(x, W1, W2):
    off = CT0 // BT
    return pl.pallas_call(
        _gate_topk_body,
        grid=(CT1 // BT,),
        in_specs=[
            pl.BlockSpec((BT, D), lambda i: (off + i, 0)),
            pl.BlockSpec((D, E), lambda i: (0, 0)),
            pl.BlockSpec((E, E), lambda i: (0, 0)),
        ],
        out_specs=[
            pl.BlockSpec((BT, K), lambda i: (i, 0)),
            pl.BlockSpec((BT, K), lambda i: (i, 0)),
        ],
        out_shape=(
            jax.ShapeDtypeStruct((CT1, K), jnp.int32),
            jax.ShapeDtypeStruct((CT1, K), jnp.float32),
        ),
    )(x, W1, W2)


def _gate_logits(x, W1, W2p):
    return pl.pallas_call(
        _gate_body,
        grid=(CT0 // BT,),
        in_specs=[
            pl.BlockSpec((BT, D), lambda i: (i, 0)),
            pl.BlockSpec((D, E), lambda i: (0, 0)),
            pl.BlockSpec((E, EP), lambda i: (0, 0)),
        ],
        out_specs=pl.BlockSpec((BT, EP), lambda i: (i, 0)),
        out_shape=jax.ShapeDtypeStruct((CT0, EP), jnp.float32),
    )(x, W1, W2p)


def _topk_body(lg_hbm, oi_hbm, os_hbm, lg_v, oi_v, os_v):
    wid = lax.axis_index("s") * 2 + lax.axis_index("c")
    pltpu.sync_copy(lg_hbm.at[pl.ds(wid * TPW, TPW)], lg_v)

    lane = lax.iota(jnp.int32, L)

    def group_body(g, _):
        rows_ab = [2 * g * L + lane, (2 * g + 1) * L + lane]

        neg_inf = jnp.full((L,), -jnp.inf, jnp.float32)
        zero_i = jnp.zeros((L,), jnp.int32)
        init = tuple(neg_inf for _ in range(K)) + tuple(zero_i for _ in range(K))

        def expert_body(e, carry):
            ca, cb = carry
            iv0 = jnp.full((L,), 0, jnp.int32) + e
            out = []
            for rows, c in ((rows_ab[0], ca), (rows_ab[1], cb)):
                ts, is_ = c[:K], c[K:]
                v = plsc.load_gather(lg_v, [rows, iv0])
                iv = iv0
                ins = jnp.zeros((L,), jnp.bool_)
                new_ts, new_is = [], []
                for r in range(K):
                    gt = v > ts[r]
                    cond = jnp.logical_or(ins, gt)
                    new_ts.append(jnp.maximum(v, ts[r]))
                    v = jnp.minimum(v, ts[r])
                    new_is.append(jnp.where(cond, iv, is_[r]))
                    iv = jnp.where(cond, is_[r], iv)
                    ins = cond
                out.append(tuple(new_ts) + tuple(new_is))
            return tuple(out)

        ca, cb = lax.fori_loop(0, E, expert_body, (init, init))

        for rows, c in ((rows_ab[0], ca), (rows_ab[1], cb)):
            ts, is_ = c[:K], c[K:]
            rows8 = rows * K
            exps = [jnp.exp(t - ts[0]) for t in ts]
            s = exps[0]
            for r in range(1, K):
                s = s + exps[r]
            inv = 1.0 / s
            for r in range(K):
                plsc.store_scatter(oi_v, [rows8 + r], is_[r])
                plsc.store_scatter(os_v, [rows8 + r], exps[r] * inv)
        return _

    lax.fori_loop(0, NG // 2, group_body, None)

    obase = wid * (TPW * K)
    pltpu.sync_copy(oi_v, oi_hbm.at[pl.ds(obase, TPW * K)])
    pltpu.sync_copy(os_v, os_hbm.at[pl.ds(obase, TPW * K)])


@functools.cache
def _topk_sc():
    return pl.kernel(
        _topk_body,
        out_type=(
            jax.ShapeDtypeStruct((CT0 * K,), jnp.int32),
            jax.ShapeDtypeStruct((CT0 * K,), jnp.float32),
        ),
        mesh=plsc.VectorSubcoreMesh(core_axis_name="c", subcore_axis_name="s"),
        compiler_params=pltpu.CompilerParams(needs_layout_passes=False),
        scratch_types=[
            pltpu.VMEM((TPW, EP), jnp.float32),
            pltpu.VMEM((TPW * K,), jnp.int32),
            pltpu.VMEM((TPW * K,), jnp.float32),
        ],
    )


def kernel(x, W1, W2):
    W2p = jnp.pad(W2, ((0, 0), (0, EP - E)))
    logits0 = _gate_logits(x, W1, W2p)
    idx0_flat, scr0_flat = _topk_sc()(logits0)
    idx1, scr1 = _gate_topk(x, W1, W2)
    idx = jnp.concatenate([idx0_flat.reshape(CT0, K), idx1], 0)
    scr = jnp.concatenate([scr0_flat.reshape(CT0, K), scr1], 0)
    return idx, scr


# speedup vs baseline: 1.0514x; 1.0211x over previous
import functools

import jax
import jax.numpy as jnp
from jax import lax
from jax.experimental import pallas as pl
from jax.experimental.pallas import tpu as pltpu
from jax.experimental.pallas import tpu_sc as plsc

E = 64
EP = 128
K = 8
D = 4096
T = 8192

BT = 512
CT0 = 6144
CT1 = T - CT0
NW = 32
TPW = CT0 // NW
NG = TPW // 16
L = 16


def _gate_body(x_ref, w1_ref, w2_ref, out_ref):
    h = jnp.tanh(jnp.dot(x_ref[...], w1_ref[...]))
    out_ref[...] = jnp.dot(h, w2_ref[...])


def _gate_topk_body(x_ref, w1_ref, w2_ref, oi_ref, os_ref):
    h = jnp.tanh(jnp.dot(x_ref[...], w1_ref[...]))
    lg = jnp.dot(h, w2_ref[...])
    cols = lax.broadcasted_iota(jnp.int32, (BT, E), 1)
    vals, idxs = [], []
    for r in range(K):
        m = jnp.max(lg, axis=1, keepdims=True)
        sel = jnp.min(jnp.where(lg == m, cols, E), axis=1, keepdims=True)
        vals.append(m)
        idxs.append(sel)
        lg = jnp.where(cols == sel, -jnp.inf, lg)
    exps = [jnp.exp(v - vals[0]) for v in vals]
    s = exps[0]
    for r in range(1, K):
        s = s + exps[r]
    inv = 1.0 / s
    oi_ref[...] = jnp.concatenate(idxs, axis=1)
    os_ref[...] = jnp.concatenate([e * inv for e in exps], axis=1)


def _gate_topk(x, W1, W2):
    off = CT0 // BT
    return pl.pallas_call(
        _gate_topk_body,
        grid=(CT1 // BT,),
        in_specs=[
            pl.BlockSpec((BT, D), lambda i: (off + i, 0)),
            pl.BlockSpec((D, E), lambda i: (0, 0)),
            pl.BlockSpec((E, E), lambda i: (0, 0)),
        ],
        out_specs=[
            pl.BlockSpec((BT, K), lambda i: (i, 0)),
            pl.BlockSpec((BT, K), lambda i: (i, 0)),
        ],
        out_shape=(
            jax.ShapeDtypeStruct((CT1, K), jnp.int32),
            jax.ShapeDtypeStruct((CT1, K), jnp.float32),
        ),
    )(x, W1, W2)


def _gate_logits(x, W1, W2p):
    return pl.pallas_call(
        _gate_body,
        grid=(CT0 // BT,),
        in_specs=[
            pl.BlockSpec((BT, D), lambda i: (i, 0)),
            pl.BlockSpec((D, E), lambda i: (0, 0)),
            pl.BlockSpec((E, EP), lambda i: (0, 0)),
        ],
        out_specs=pl.BlockSpec((BT, EP), lambda i: (i, 0)),
        out_shape=jax.ShapeDtypeStruct((CT0, EP), jnp.float32),
    )(x, W1, W2p)


def _topk_body(lg_hbm, oi_hbm, os_hbm, lg_v, oi_v, os_v):
    wid = lax.axis_index("s") * 2 + lax.axis_index("c")
    pltpu.sync_copy(lg_hbm.at[pl.ds(wid * TPW, TPW)], lg_v)

    lane = lax.iota(jnp.int32, L)

    def group_body(g, _):
        rows_ab = [2 * g * L + lane, (2 * g + 1) * L + lane]

        neg_inf = jnp.full((L,), -jnp.inf, jnp.float32)
        zero_i = jnp.zeros((L,), jnp.int32)
        init = tuple(neg_inf for _ in range(K)) + tuple(zero_i for _ in range(K))

        def expert_body(e, carry):
            ca, cb = carry
            iv0 = jnp.full((L,), 0, jnp.int32) + e
            out = []
            for rows, c in ((rows_ab[0], ca), (rows_ab[1], cb)):
                ts, is_ = c[:K], c[K:]
                v = plsc.load_gather(lg_v, [rows, iv0])
                iv = iv0
                ins = jnp.zeros((L,), jnp.bool_)
                new_ts, new_is = [], []
                for r in range(K):
                    gt = v > ts[r]
                    cond = jnp.logical_or(ins, gt)
                    new_ts.append(jnp.maximum(v, ts[r]))
                    v = jnp.minimum(v, ts[r])
                    new_is.append(jnp.where(cond, iv, is_[r]))
                    iv = jnp.where(cond, is_[r], iv)
                    ins = cond
                out.append(tuple(new_ts) + tuple(new_is))
            return tuple(out)

        ca, cb = lax.fori_loop(0, E, expert_body, (init, init))

        for rows, c in ((rows_ab[0], ca), (rows_ab[1], cb)):
            ts, is_ = c[:K], c[K:]
            exps = [jnp.exp(t - ts[0]) for t in ts]
            s = exps[0]
            for r in range(1, K):
                s = s + exps[r]
            inv = 1.0 / s
            for r in range(K):
                colr = jnp.full((L,), r, jnp.int32)
                plsc.store_scatter(oi_v, [rows, colr], is_[r])
                plsc.store_scatter(os_v, [rows, colr], exps[r] * inv)
        return _

    lax.fori_loop(0, NG // 2, group_body, None)

    pltpu.sync_copy(oi_v, oi_hbm.at[pl.ds(wid * TPW, TPW)])
    pltpu.sync_copy(os_v, os_hbm.at[pl.ds(wid * TPW, TPW)])


@functools.cache
def _topk_sc():
    return pl.kernel(
        _topk_body,
        out_type=(
            jax.ShapeDtypeStruct((CT0, K), jnp.int32),
            jax.ShapeDtypeStruct((CT0, K), jnp.float32),
        ),
        mesh=plsc.VectorSubcoreMesh(core_axis_name="c", subcore_axis_name="s"),
        compiler_params=pltpu.CompilerParams(needs_layout_passes=False),
        scratch_types=[
            pltpu.VMEM((TPW, EP), jnp.float32),
            pltpu.VMEM((TPW, K), jnp.int32),
            pltpu.VMEM((TPW, K), jnp.float32),
        ],
    )


def kernel(x, W1, W2):
    W2p = jnp.pad(W2, ((0, 0), (0, EP - E)))
    logits0 = _gate_logits(x, W1, W2p)
    idx0, scr0 = _topk_sc()(logits0)
    idx1, scr1 = _gate_topk(x, W1, W2)
    idx = jnp.concatenate([idx0, idx1], 0)
    scr = jnp.concatenate([scr0, scr1], 0)
    return idx, scr


# speedup vs baseline: 1.0599x; 1.0081x over previous
import functools

import jax
import jax.numpy as jnp
from jax import lax
from jax.experimental import pallas as pl
from jax.experimental.pallas import tpu as pltpu
from jax.experimental.pallas import tpu_sc as plsc

E = 64
EP = 128
K = 8
D = 4096
T = 8192

BT = 512
CT0 = 6144
CT1 = T - CT0
NW = 32
TPW = CT0 // NW
NG = TPW // 16
L = 16


def _gate_body(x_ref, w1_ref, w2_ref, out_ref):
    h = jnp.tanh(jnp.dot(x_ref[...], w1_ref[...]))
    out_ref[...] = jnp.dot(h, w2_ref[...])


def _gate_topk_body(x_ref, w1_ref, w2_ref, oi_ref, os_ref):
    h = jnp.tanh(jnp.dot(x_ref[...], w1_ref[...]))
    lg = jnp.dot(h, w2_ref[...])
    cols = lax.broadcasted_iota(jnp.int32, (BT, E), 1)
    vals, idxs = [], []
    for r in range(K):
        m = jnp.max(lg, axis=1, keepdims=True)
        sel = jnp.min(jnp.where(lg == m, cols, E), axis=1, keepdims=True)
        vals.append(m)
        idxs.append(sel)
        lg = jnp.where(cols == sel, -jnp.inf, lg)
    exps = [jnp.exp(v - vals[0]) for v in vals]
    s = exps[0]
    for r in range(1, K):
        s = s + exps[r]
    inv = 1.0 / s
    oi_ref[...] = jnp.concatenate(idxs, axis=1)
    os_ref[...] = jnp.concatenate([e * inv for e in exps], axis=1)


def _gate_topk(x, W1, W2):
    off = CT0 // BT
    return pl.pallas_call(
        _gate_topk_body,
        grid=(CT1 // BT,),
        in_specs=[
            pl.BlockSpec((BT, D), lambda i: (off + i, 0)),
            pl.BlockSpec((D, E), lambda i: (0, 0)),
            pl.BlockSpec((E, E), lambda i: (0, 0)),
        ],
        out_specs=[
            pl.BlockSpec((BT, K), lambda i: (i, 0)),
            pl.BlockSpec((BT, K), lambda i: (i, 0)),
        ],
        out_shape=(
            jax.ShapeDtypeStruct((CT1, K), jnp.int32),
            jax.ShapeDtypeStruct((CT1, K), jnp.float32),
        ),
    )(x, W1, W2)


def _gate_logits(x, W1, W2):
    return pl.pallas_call(
        _gate_body,
        grid=(CT0 // BT,),
        in_specs=[
            pl.BlockSpec((BT, D), lambda i: (i, 0)),
            pl.BlockSpec((D, E), lambda i: (0, 0)),
            pl.BlockSpec((E, E), lambda i: (0, 0)),
        ],
        out_specs=pl.BlockSpec((BT, E), lambda i: (i, 0)),
        out_shape=jax.ShapeDtypeStruct((CT0, E), jnp.float32),
    )(x, W1, W2)


def _topk_body(lg_hbm, oi_hbm, os_hbm, lg_v, oi_v, os_v):
    wid = lax.axis_index("s") * 2 + lax.axis_index("c")
    pltpu.sync_copy(lg_hbm.at[pl.ds(wid * TPW, TPW)], lg_v)

    lane = lax.iota(jnp.int32, L)

    def group_body(g, _):
        rows_ab = [2 * g * L + lane, (2 * g + 1) * L + lane]

        neg_inf = jnp.full((L,), -jnp.inf, jnp.float32)
        zero_i = jnp.zeros((L,), jnp.int32)
        init = tuple(neg_inf for _ in range(K)) + tuple(zero_i for _ in range(K))

        def expert_body(e, carry):
            ca, cb = carry
            iv0 = jnp.full((L,), 0, jnp.int32) + e
            out = []
            for rows, c in ((rows_ab[0], ca), (rows_ab[1], cb)):
                ts, is_ = c[:K], c[K:]
                v = plsc.load_gather(lg_v, [rows, iv0])
                iv = iv0
                ins = jnp.zeros((L,), jnp.bool_)
                new_ts, new_is = [], []
                for r in range(K):
                    gt = v > ts[r]
                    cond = jnp.logical_or(ins, gt)
                    new_ts.append(jnp.maximum(v, ts[r]))
                    v = jnp.minimum(v, ts[r])
                    new_is.append(jnp.where(cond, iv, is_[r]))
                    iv = jnp.where(cond, is_[r], iv)
                    ins = cond
                out.append(tuple(new_ts) + tuple(new_is))
            return tuple(out)

        ca, cb = lax.fori_loop(0, E, expert_body, (init, init))

        for rows, c in ((rows_ab[0], ca), (rows_ab[1], cb)):
            ts, is_ = c[:K], c[K:]
            exps = [jnp.exp(t - ts[0]) for t in ts]
            s = exps[0]
            for r in range(1, K):
                s = s + exps[r]
            inv = 1.0 / s
            for r in range(K):
                colr = jnp.full((L,), r, jnp.int32)
                plsc.store_scatter(oi_v, [rows, colr], is_[r])
                plsc.store_scatter(os_v, [rows, colr], exps[r] * inv)
        return _

    lax.fori_loop(0, NG // 2, group_body, None)

    pltpu.sync_copy(oi_v, oi_hbm.at[pl.ds(wid * TPW, TPW)])
    pltpu.sync_copy(os_v, os_hbm.at[pl.ds(wid * TPW, TPW)])


@functools.cache
def _topk_sc():
    return pl.kernel(
        _topk_body,
        out_type=(
            jax.ShapeDtypeStruct((CT0, K), jnp.int32),
            jax.ShapeDtypeStruct((CT0, K), jnp.float32),
        ),
        mesh=plsc.VectorSubcoreMesh(core_axis_name="c", subcore_axis_name="s"),
        compiler_params=pltpu.CompilerParams(needs_layout_passes=False),
        scratch_types=[
            pltpu.VMEM((TPW, E), jnp.float32),
            pltpu.VMEM((TPW, K), jnp.int32),
            pltpu.VMEM((TPW, K), jnp.float32),
        ],
    )


def kernel(x, W1, W2):
    logits0 = _gate_logits(x, W1, W2)
    idx0, scr0 = _topk_sc()(logits0)
    idx1, scr1 = _gate_topk(x, W1, W2)
    idx = jnp.concatenate([idx0, idx1], 0)
    scr = jnp.concatenate([scr0, scr1], 0)
    return idx, scr
